# baseline (speedup 1.0000x reference)
import jax
import jax.numpy as jnp
from jax import lax
from jax.experimental import pallas as pl
from jax.experimental.pallas import tpu as pltpu
from jax.experimental.pallas import tpu_sc as plsc

B, K, D = 1024, 50, 32
N = 1000000
KPI = 56
KPA = 64
NC, NS = 2, 16
NW = NC * NS
BPW = B // NW
HALF = D // 2


def _dis_loss_body(emb_hbm, table_hbm, attr_hbm, idx_hbm, out_hbm,
                   embb_v, attr_v, idx_v, rows_v, o_v, sem):
    wid = lax.axis_index("s") * NC + lax.axis_index("c")
    pltpu.sync_copy(emb_hbm.at[pl.ds(wid * (BPW * D), BPW * D)], embb_v)
    pltpu.sync_copy(attr_hbm.at[pl.ds(wid * (BPW * KPA), BPW * KPA)], attr_v)
    pltpu.sync_copy(idx_hbm.at[pl.ds(wid * (BPW * KPI), BPW * KPI)], idx_v)

    for b in range(BPW):
        pltpu.async_copy(table_hbm.at[idx_v.at[pl.ds(b * KPI, KPI)]],
                         rows_v.at[b], sem)
    for b in range(BPW):
        pltpu.make_async_copy(
            table_hbm.at[idx_v.at[pl.ds(b * KPI, KPI)]],
            rows_v.at[b], sem).wait()

    def b_loop(b, carry):
        acc_lo, acc_hi = carry
        x_lo = embb_v[pl.ds(b * D, HALF)]
        x_hi = embb_v[pl.ds(b * D + HALF, HALF)]
        for g2 in range(KPA // HALF):
            av = attr_v[pl.ds(b * KPA + g2 * HALF, HALF)]
            for kk in range(HALF):
                k = g2 * HALF + kk
                if k >= KPI:
                    break
                a = av[kk]
                d_lo = rows_v[b, k, 0:HALF] - x_lo
                d_hi = rows_v[b, k, HALF:D] - x_hi
                acc_lo = acc_lo + a * (d_lo * d_lo)
                acc_hi = acc_hi + a * (d_hi * d_hi)
        return (acc_lo, acc_hi)

    z = jnp.zeros((HALF,), jnp.float32)
    acc_lo, acc_hi = lax.fori_loop(0, BPW, b_loop, (z, z))
    o_v[...] = acc_lo + acc_hi
    pltpu.sync_copy(o_v, out_hbm.at[pl.ds(wid * HALF, HALF)])


def kernel(emb_batch, embedding, attr_sim, indices, beta):
    del beta
    idx_p = jnp.pad(indices, ((0, 0), (0, KPI - K))).reshape(-1)
    attr_p = jnp.pad(attr_sim, ((0, 0), (0, KPA - K))).reshape(-1)
    emb_flat = emb_batch.reshape(-1)
    mesh = plsc.VectorSubcoreMesh(core_axis_name="c", subcore_axis_name="s")
    out = pl.kernel(
        _dis_loss_body,
        out_type=jax.ShapeDtypeStruct((NW * HALF,), jnp.float32),
        mesh=mesh,
        compiler_params=pltpu.CompilerParams(use_tc_tiling_on_sc=False),
        scratch_types=[
            pltpu.VMEM((BPW * D,), jnp.float32),
            pltpu.VMEM((BPW * KPA,), jnp.float32),
            pltpu.VMEM((BPW * KPI,), jnp.int32),
            pltpu.VMEM((BPW, KPI, D), jnp.float32),
            pltpu.VMEM((HALF,), jnp.float32),
            pltpu.SemaphoreType.DMA,
        ],
    )(emb_flat, embedding, attr_p, idx_p)
    return jnp.sum(out) / jnp.float32(B)
